# baseline (speedup 1.0000x reference)
import functools

import jax
import jax.numpy as jnp
from jax import lax
from jax.experimental import pallas as pl
from jax.experimental.pallas import tpu as pltpu
from jax.experimental.pallas import tpu_sc as plsc

_NC = 2
_NS = 16
_NW = _NC * _NS
_L = 16
_CHUNK = 128


def _tree_sum(parts):
    while len(parts) > 1:
        parts = [a + b for a, b in zip(parts[::2], parts[1::2])]
    return parts[0]


def _vperm(x, p):
    dn = lax.GatherDimensionNumbers(
        offset_dims=(), collapsed_slice_dims=(0,), start_index_map=(0,))
    return lax.gather(x, p[:, None], dn, slice_sizes=(1,),
                      mode=lax.GatherScatterMode.PROMISE_IN_BOUNDS)


def _lane_sum(x, perms):
    for p in perms:
        x = x + _vperm(x, p)
    return x


def _pair_sum(a, b, perms, lo_mask):
    a2 = a + _vperm(a, perms[0])
    b2 = b + _vperm(b, perms[0])
    c = jnp.where(lo_mask, a2, _vperm(b2, perms[0]))
    for p in perms[1:]:
        c = c + _vperm(c, p)
    return c


def _rsqrt_scalar(x):
    i = lax.bitcast_convert_type(x, jnp.int32)
    i = 0x5F3759DF - lax.shift_right_logical(i, 1)
    y = lax.bitcast_convert_type(i, jnp.float32)
    for _ in range(3):
        y = y * (1.5 - 0.5 * x * y * y)
    return y


def _score_scalar(qq_s, s_s, ss_s):
    rn = _rsqrt_scalar(jnp.maximum(ss_s, 1e-24))
    val = jnp.maximum(qq_s + 1.0 - 2.0 * s_s * rn, 0.0)
    return -(val * _rsqrt_scalar(jnp.maximum(val, 1e-30)))


def _sc_build(batch, dim, rpe):
    per_w = batch // _NW
    rows_w = per_w * rpe
    n_chunks = rows_w // _CHUNK
    n_pairs = n_chunks // 2
    dc = dim // _L
    mesh = plsc.VectorSubcoreMesh(core_axis_name="c", subcore_axis_name="s")

    @functools.partial(
        pl.kernel,
        out_type=jax.ShapeDtypeStruct((batch * rpe,), jnp.float32),
        mesh=mesh,
        scratch_types=[
            pltpu.VMEM((per_w,), jnp.int32),
            pltpu.VMEM((per_w,), jnp.int32),
            pltpu.VMEM((per_w, dim), jnp.float32),
            pltpu.VMEM((per_w, dim), jnp.float32),
            pltpu.VMEM((per_w, dim), jnp.float32),
            pltpu.VMEM((per_w * _L,), jnp.float32),
            pltpu.VMEM((rows_w,), jnp.int32),
            pltpu.VMEM((rows_w,), jnp.int32),
            pltpu.VMEM((2, _CHUNK, dim), jnp.float32),
            pltpu.VMEM((2, _CHUNK), jnp.float32),
            pltpu.SemaphoreType.DMA,
            pltpu.SemaphoreType.DMA,
            pltpu.SemaphoreType.DMA,
            pltpu.SemaphoreType.DMA,
            pltpu.SemaphoreType.DMA,
        ],
    )
    def k(ent_hbm, hidx_hbm, rel_hbm, ridx_hbm, gidx_hbm, eid_hbm,
          s_out,
          hidx_v, ridx_v, hrows_v, rrows_v, q_tab, qq_v,
          gidx_v, eid_v, rows_v, a_v, sem,
          gsem0, gsem1, osem0, osem1):
        wid = lax.axis_index("s") * _NC + lax.axis_index("c")
        ebase = wid * per_w
        rbase_w = wid * rows_w
        lanes = lax.iota(jnp.int32, _L)
        perms = [lanes ^ k for k in (8, 4, 2, 1)]
        lo_mask = lanes < (_L // 2)
        pair_masks = [(lanes & (_L // 2 - 1)) == l for l in range(_L // 2)]

        pltpu.sync_copy(gidx_hbm.at[pl.ds(rbase_w, rows_w)], gidx_v)
        pltpu.sync_copy(eid_hbm.at[pl.ds(rbase_w, rows_w)], eid_v)

        def gather_of(ch, p, gsem):
            return pltpu.async_copy(
                ent_hbm.at[gidx_v.at[pl.ds(ch * _CHUNK, _CHUNK)]],
                rows_v.at[p], gsem)

        gather_of(0, 0, gsem0)
        gather_of(1, 1, gsem1)

        pltpu.sync_copy(hidx_hbm.at[pl.ds(ebase, per_w)], hidx_v)
        pltpu.sync_copy(ridx_hbm.at[pl.ds(ebase, per_w)], ridx_v)
        pltpu.async_copy(ent_hbm.at[hidx_v], hrows_v, sem).wait()
        pltpu.async_copy(rel_hbm.at[ridx_v], rrows_v, sem).wait()

        def stage_a(e, carry):
            hch = [hrows_v[e, pl.ds(c * _L, _L)] for c in range(dc)]
            rch = [rrows_v[e, pl.ds(c * _L, _L)] for c in range(dc)]
            hh = _lane_sum(_tree_sum([h * h for h in hch]), perms)
            hr = _lane_sum(_tree_sum([h * r for h, r in zip(hch, rch)]),
                           perms)
            hh_s = hh[0]
            i = lax.bitcast_convert_type(hh_s, jnp.int32)
            i = 0x5F3759DF - lax.shift_right_logical(i, 1)
            y = lax.bitcast_convert_type(i, jnp.float32)
            for _ in range(3):
                y = y * (1.5 - 0.5 * hh_s * y * y)
            for c in range(dc):
                q_tab[e, pl.ds(c * _L, _L)] = hch[c] * y + rch[c]
            qq_v[pl.ds(e * _L, _L)] = 2.0 + 2.0 * hr * y
            return carry

        lax.fori_loop(0, per_w, stage_a, 0)

        def wait_gather(ch, p, gsem):
            pltpu.make_async_copy(
                ent_hbm.at[gidx_v.at[pl.ds(ch * _CHUNK, _CHUNK)]],
                rows_v.at[p], gsem).wait()

        def compute_chunk(ch, p):
            def group_body(g, carry2):
                ev = eid_v[pl.ds(ch * _CHUNK + g * _L, _L)]
                res_s = jnp.zeros((_L,), jnp.float32)
                for l in range(_L // 2):
                    rx = g * _L + l
                    ry = rx + _L // 2
                    ex = ev[l]
                    ey = ev[l + _L // 2]
                    nx = [rows_v[p, rx, pl.ds(c * _L, _L)]
                          for c in range(dc)]
                    ny = [rows_v[p, ry, pl.ds(c * _L, _L)]
                          for c in range(dc)]
                    qx = [q_tab[ex, pl.ds(c * _L, _L)] for c in range(dc)]
                    qy = [q_tab[ey, pl.ds(c * _L, _L)] for c in range(dc)]
                    px = _tree_sum([n * q for n, q in zip(nx, qx)])
                    py = _tree_sum([n * q for n, q in zip(ny, qy)])
                    pxx = _tree_sum([n * n for n in nx])
                    pyy = _tree_sum([n * n for n in ny])
                    sv = _pair_sum(px, py, perms, lo_mask)
                    ssv = _pair_sum(pxx, pyy, perms, lo_mask)
                    qqx = qq_v[pl.ds(ex * _L, _L)][0]
                    qqy = qq_v[pl.ds(ey * _L, _L)][0]
                    scx = _score_scalar(qqx, sv[0], ssv[0])
                    scy = _score_scalar(qqy, sv[_L // 2], ssv[_L // 2])
                    res_s = jnp.where(lanes == l, scx, res_s)
                    res_s = jnp.where(lanes == l + _L // 2, scy, res_s)
                a_v[p, pl.ds(g * _L, _L)] = res_s
                return carry2

            lax.fori_loop(0, _CHUNK // _L, group_body, 0)

        def writeout(ch, p, osem):
            rb = rbase_w + ch * _CHUNK
            pltpu.async_copy(a_v.at[p], s_out.at[pl.ds(rb, _CHUNK)], osem)

        def drain_out(ch, p, osem):
            rb = rbase_w + ch * _CHUNK
            pltpu.make_async_copy(
                a_v.at[p], s_out.at[pl.ds(rb, _CHUNK)], osem).wait()

        def pair_body(i, carry):
            c0 = 2 * i
            wait_gather(c0, 0, gsem0)

            @pl.when(i > 0)
            def _():
                drain_out(c0 - 2, 0, osem0)

            compute_chunk(c0, 0)
            writeout(c0, 0, osem0)
            gather_of(c0 + 2, 0, gsem0)

            wait_gather(c0 + 1, 1, gsem1)

            @pl.when(i > 0)
            def _():
                drain_out(c0 - 1, 1, osem1)

            compute_chunk(c0 + 1, 1)
            writeout(c0 + 1, 1, osem1)
            gather_of((c0 + 3) % n_chunks, 1, gsem1)
            return carry

        lax.fori_loop(0, n_pairs, pair_body, 0)
        wait_gather(n_chunks - 1, 0, gsem0)
        drain_out(n_chunks - 3, 0, osem0)
        compute_chunk(n_chunks - 1, 0)
        writeout(n_chunks - 1, 0, osem0)
        wait_gather(0, 1, gsem1)
        drain_out(n_chunks - 2, 1, osem1)
        drain_out(n_chunks - 1, 0, osem0)

    return k


def _tc_split_body(sm_ref, pos_ref, negs_ref):
    sm = sm_ref[...]
    pos_ref[...] = sm[:, 0]
    negs_ref[...] = sm[:, 1:]


def _tc_split(sm, batch, n_neg, rpe):
    return pl.pallas_call(
        _tc_split_body,
        grid=(1,),
        in_specs=[pl.BlockSpec((batch, rpe), lambda i: (0, 0))],
        out_specs=[
            pl.BlockSpec((batch,), lambda i: (0,)),
            pl.BlockSpec((batch, n_neg), lambda i: (0, 0)),
        ],
        out_shape=[
            jax.ShapeDtypeStruct((batch,), jnp.float32),
            jax.ShapeDtypeStruct((batch, n_neg), jnp.float32),
        ],
    )(sm)


def kernel(head_idx, relation_idx, tail_idx, negative_idx, entity_emb, relation_emb):
    batch = head_idx.shape[0]
    n_neg = negative_idx.shape[1]
    dim = entity_emb.shape[1]
    rpe = n_neg + 1
    per_w = batch // _NW
    gidx = jnp.concatenate([tail_idx[:, None], negative_idx],
                           axis=1).reshape(-1)
    eids = jnp.repeat(jnp.arange(batch, dtype=jnp.int32) % per_w, rpe)
    sc = _sc_build(batch, dim, rpe)
    scores = sc(entity_emb, head_idx, relation_emb, relation_idx, gidx, eids)
    pos, negs = _tc_split(scores.reshape(batch, rpe), batch, n_neg, rpe)
    return pos, negs


# speedup vs baseline: 1.7923x; 1.7923x over previous
import functools

import jax
import jax.numpy as jnp
from jax import lax
from jax.experimental import pallas as pl
from jax.experimental.pallas import tpu as pltpu
from jax.experimental.pallas import tpu_sc as plsc

_NC = 2
_NS = 16
_NW = _NC * _NS
_L = 16
_CHUNK = 128


def _tree_sum(parts):
    while len(parts) > 1:
        parts = [a + b for a, b in zip(parts[::2], parts[1::2])]
    return parts[0]


def _vperm(x, p):
    dn = lax.GatherDimensionNumbers(
        offset_dims=(), collapsed_slice_dims=(0,), start_index_map=(0,))
    return lax.gather(x, p[:, None], dn, slice_sizes=(1,),
                      mode=lax.GatherScatterMode.PROMISE_IN_BOUNDS)


def _lane_sum(x, perms):
    for p in perms:
        x = x + _vperm(x, p)
    return x


def _pair_sum(a, b, perms, lo_mask):
    a2 = a + _vperm(a, perms[0])
    b2 = b + _vperm(b, perms[0])
    c = jnp.where(lo_mask, a2, _vperm(b2, perms[0]))
    for p in perms[1:]:
        c = c + _vperm(c, p)
    return c


def _sc_build(batch, dim, rpe):
    per_w = batch // _NW
    rows_w = per_w * rpe
    n_chunks = rows_w // _CHUNK
    n_pairs = n_chunks // 2
    dc = dim // _L
    mesh = plsc.VectorSubcoreMesh(core_axis_name="c", subcore_axis_name="s")

    @functools.partial(
        pl.kernel,
        out_type=[
            jax.ShapeDtypeStruct((batch * rpe,), jnp.float32),
            jax.ShapeDtypeStruct((batch * rpe,), jnp.float32),
            jax.ShapeDtypeStruct((batch * _L,), jnp.float32),
        ],
        mesh=mesh,
        scratch_types=[
            pltpu.VMEM((per_w,), jnp.int32),
            pltpu.VMEM((per_w,), jnp.int32),
            pltpu.VMEM((per_w, dim), jnp.float32),
            pltpu.VMEM((per_w, dim), jnp.float32),
            pltpu.VMEM((per_w, dim), jnp.float32),
            pltpu.VMEM((per_w * _L,), jnp.float32),
            pltpu.VMEM((rows_w,), jnp.int32),
            pltpu.VMEM((rows_w,), jnp.int32),
            pltpu.VMEM((2, _CHUNK, dim), jnp.float32),
            pltpu.VMEM((2, _CHUNK), jnp.float32),
            pltpu.VMEM((2, _CHUNK), jnp.float32),
            pltpu.SemaphoreType.DMA,
            pltpu.SemaphoreType.DMA,
            pltpu.SemaphoreType.DMA,
            pltpu.SemaphoreType.DMA,
            pltpu.SemaphoreType.DMA,
        ],
    )
    def k(ent_hbm, hidx_hbm, rel_hbm, ridx_hbm, gidx_hbm, eid_hbm,
          s_out, ss_out, qq_out,
          hidx_v, ridx_v, hrows_v, rrows_v, q_tab, qq_v,
          gidx_v, eid_v, rows_v, a_v, ss_v, sem,
          gsem0, gsem1, osem0, osem1):
        wid = lax.axis_index("s") * _NC + lax.axis_index("c")
        ebase = wid * per_w
        rbase_w = wid * rows_w
        lanes = lax.iota(jnp.int32, _L)
        perms = [lanes ^ k for k in (8, 4, 2, 1)]
        lo_mask = lanes < (_L // 2)
        pair_masks = [(lanes & (_L // 2 - 1)) == l for l in range(_L // 2)]

        pltpu.sync_copy(gidx_hbm.at[pl.ds(rbase_w, rows_w)], gidx_v)
        pltpu.sync_copy(eid_hbm.at[pl.ds(rbase_w, rows_w)], eid_v)

        def gather_of(ch, p, gsem):
            return pltpu.async_copy(
                ent_hbm.at[gidx_v.at[pl.ds(ch * _CHUNK, _CHUNK)]],
                rows_v.at[p], gsem)

        gather_of(0, 0, gsem0)
        gather_of(1, 1, gsem1)

        pltpu.sync_copy(hidx_hbm.at[pl.ds(ebase, per_w)], hidx_v)
        pltpu.sync_copy(ridx_hbm.at[pl.ds(ebase, per_w)], ridx_v)
        pltpu.async_copy(ent_hbm.at[hidx_v], hrows_v, sem).wait()
        pltpu.async_copy(rel_hbm.at[ridx_v], rrows_v, sem).wait()

        def stage_a(e, carry):
            hch = [hrows_v[e, pl.ds(c * _L, _L)] for c in range(dc)]
            rch = [rrows_v[e, pl.ds(c * _L, _L)] for c in range(dc)]
            hh = _lane_sum(_tree_sum([h * h for h in hch]), perms)
            hr = _lane_sum(_tree_sum([h * r for h, r in zip(hch, rch)]),
                           perms)
            hh_s = hh[0]
            i = lax.bitcast_convert_type(hh_s, jnp.int32)
            i = 0x5F3759DF - lax.shift_right_logical(i, 1)
            y = lax.bitcast_convert_type(i, jnp.float32)
            for _ in range(3):
                y = y * (1.5 - 0.5 * hh_s * y * y)
            for c in range(dc):
                q_tab[e, pl.ds(c * _L, _L)] = hch[c] * y + rch[c]
            qq_v[pl.ds(e * _L, _L)] = 2.0 + 2.0 * hr * y
            return carry

        lax.fori_loop(0, per_w, stage_a, 0)
        pltpu.sync_copy(qq_v, qq_out.at[pl.ds(ebase * _L, per_w * _L)])

        def wait_gather(ch, p, gsem):
            pltpu.make_async_copy(
                ent_hbm.at[gidx_v.at[pl.ds(ch * _CHUNK, _CHUNK)]],
                rows_v.at[p], gsem).wait()

        def compute_chunk(ch, p):
            def group_body(g, carry2):
                ev = eid_v[pl.ds(ch * _CHUNK + g * _L, _L)]
                res_s = jnp.zeros((_L,), jnp.float32)
                res_ss = jnp.zeros((_L,), jnp.float32)
                for l in range(_L // 2):
                    rx = g * _L + l
                    ry = rx + _L // 2
                    ex = ev[l]
                    ey = ev[l + _L // 2]
                    nx = [rows_v[p, rx, pl.ds(c * _L, _L)]
                          for c in range(dc)]
                    ny = [rows_v[p, ry, pl.ds(c * _L, _L)]
                          for c in range(dc)]
                    qx = [q_tab[ex, pl.ds(c * _L, _L)] for c in range(dc)]
                    qy = [q_tab[ey, pl.ds(c * _L, _L)] for c in range(dc)]
                    px = _tree_sum([n * q for n, q in zip(nx, qx)])
                    py = _tree_sum([n * q for n, q in zip(ny, qy)])
                    pxx = _tree_sum([n * n for n in nx])
                    pyy = _tree_sum([n * n for n in ny])
                    sv = _pair_sum(px, py, perms, lo_mask)
                    ssv = _pair_sum(pxx, pyy, perms, lo_mask)
                    res_s = jnp.where(pair_masks[l], sv, res_s)
                    res_ss = jnp.where(pair_masks[l], ssv, res_ss)
                a_v[p, pl.ds(g * _L, _L)] = res_s
                ss_v[p, pl.ds(g * _L, _L)] = res_ss
                return carry2

            lax.fori_loop(0, _CHUNK // _L, group_body, 0)

        def writeout(ch, p, osem):
            rb = rbase_w + ch * _CHUNK
            pltpu.async_copy(a_v.at[p], s_out.at[pl.ds(rb, _CHUNK)], osem)
            pltpu.async_copy(ss_v.at[p], ss_out.at[pl.ds(rb, _CHUNK)], osem)

        def drain_out(ch, p, osem):
            rb = rbase_w + ch * _CHUNK
            pltpu.make_async_copy(
                a_v.at[p], s_out.at[pl.ds(rb, _CHUNK)], osem).wait()
            pltpu.make_async_copy(
                ss_v.at[p], ss_out.at[pl.ds(rb, _CHUNK)], osem).wait()

        def pair_body(i, carry):
            c0 = 2 * i
            wait_gather(c0, 0, gsem0)

            @pl.when(i > 0)
            def _():
                drain_out(c0 - 2, 0, osem0)

            compute_chunk(c0, 0)
            writeout(c0, 0, osem0)
            gather_of(c0 + 2, 0, gsem0)

            wait_gather(c0 + 1, 1, gsem1)

            @pl.when(i > 0)
            def _():
                drain_out(c0 - 1, 1, osem1)

            compute_chunk(c0 + 1, 1)
            writeout(c0 + 1, 1, osem1)
            gather_of((c0 + 3) % n_chunks, 1, gsem1)
            return carry

        lax.fori_loop(0, n_pairs, pair_body, 0)
        wait_gather(n_chunks - 1, 0, gsem0)
        drain_out(n_chunks - 3, 0, osem0)
        compute_chunk(n_chunks - 1, 0)
        writeout(n_chunks - 1, 0, osem0)
        wait_gather(0, 1, gsem1)
        drain_out(n_chunks - 2, 1, osem1)
        drain_out(n_chunks - 1, 0, osem0)

    return k


def _tc_epilogue_body(s_ref, ss_ref, qq_ref, pos_ref, negs_ref):
    s = s_ref[...]
    ss = ss_ref[...]
    qq = qq_ref[...][:, :1]
    rn = 1.0 / jnp.maximum(jnp.sqrt(ss), 1e-12)
    sc2 = jnp.maximum(qq + 1.0 - 2.0 * s * rn, 0.0)
    sc = -jnp.sqrt(sc2)
    pos_ref[...] = sc[:, 0]
    negs_ref[...] = sc[:, 1:]


def _tc_epilogue(s, ss, qq, batch, n_neg, rpe):
    blk = batch
    return pl.pallas_call(
        _tc_epilogue_body,
        grid=(batch // blk,),
        in_specs=[
            pl.BlockSpec((blk, rpe), lambda i: (i, 0)),
            pl.BlockSpec((blk, rpe), lambda i: (i, 0)),
            pl.BlockSpec((blk, _L), lambda i: (i, 0)),
        ],
        out_specs=[
            pl.BlockSpec((blk,), lambda i: (i,)),
            pl.BlockSpec((blk, n_neg), lambda i: (i, 0)),
        ],
        out_shape=[
            jax.ShapeDtypeStruct((batch,), jnp.float32),
            jax.ShapeDtypeStruct((batch, n_neg), jnp.float32),
        ],
    )(s, ss, qq)


def kernel(head_idx, relation_idx, tail_idx, negative_idx, entity_emb, relation_emb):
    batch = head_idx.shape[0]
    n_neg = negative_idx.shape[1]
    dim = entity_emb.shape[1]
    rpe = n_neg + 1
    per_w = batch // _NW
    gidx = jnp.concatenate([tail_idx[:, None], negative_idx],
                           axis=1).reshape(-1)
    eids = jnp.repeat(jnp.arange(batch, dtype=jnp.int32) % per_w, rpe)
    sc = _sc_build(batch, dim, rpe)
    s, ss, qq = sc(entity_emb, head_idx, relation_emb, relation_idx, gidx,
                   eids)
    pos, negs = _tc_epilogue(
        s.reshape(batch, rpe), ss.reshape(batch, rpe),
        qq.reshape(batch, _L), batch, n_neg, rpe)
    return pos, negs
